# baseline (speedup 1.0000x reference)
import functools

import jax
import jax.numpy as jnp
from jax import lax
from jax.experimental import pallas as pl
from jax.experimental.pallas import tpu as pltpu
from jax.experimental.pallas import tpu_sc as plsc

_CHUNK = 128


def _prep_body(ct_ref, o_ref):
    ct = ct_ref[...]
    rows = jnp.transpose(ct)
    o_ref[...] = jnp.concatenate(
        [rows, jnp.zeros_like(rows)], axis=1)


def _make_gather(V, T):
    info = plsc.get_sparse_core_info()
    NC, NS = info.num_cores, info.num_subcores
    NW = NC * NS
    b_per_w = T // NW
    n_chunks = b_per_w // _CHUNK
    mesh = plsc.VectorSubcoreMesh(core_axis_name="c", subcore_axis_name="s")

    @functools.partial(
        pl.kernel,
        mesh=mesh,
        out_type=jax.ShapeDtypeStruct((T, 128), jnp.float32),
        scratch_types=[
            pltpu.VMEM((n_chunks, _CHUNK), jnp.int32),
            pltpu.VMEM((4, _CHUNK, 128), jnp.float32),
            pltpu.SemaphoreType.DMA,
            pltpu.SemaphoreType.DMA,
            pltpu.SemaphoreType.DMA,
            pltpu.SemaphoreType.DMA,
        ],
    )
    def gather_k(pos_hbm, src_hbm, out_hbm, idx_v, rows_v, s0, s1, s2, s3):
        wid = lax.axis_index("s") * NC + lax.axis_index("c")
        base = wid * b_per_w
        pltpu.sync_copy(pos_hbm.at[wid], idx_v)
        sems = (s0, s1, s2, s3)
        nb = 4
        handles = [None] * nb
        for j in range(n_chunks):
            b = j % nb
            if handles[b] is not None:
                handles[b].wait()
                pltpu.sync_copy(
                    rows_v.at[b],
                    out_hbm.at[pl.ds(base + (j - nb) * _CHUNK, _CHUNK)],
                )
            handles[b] = pltpu.async_copy(
                src_hbm.at[idx_v.at[j]], rows_v.at[b], sems[b]
            )
        for j in range(max(0, n_chunks - nb), n_chunks):
            b = j % nb
            handles[b].wait()
            pltpu.sync_copy(
                rows_v.at[b],
                out_hbm.at[pl.ds(base + j * _CHUNK, _CHUNK)],
            )

    return gather_k


def _expand_mats():
    r = lax.broadcasted_iota(jnp.int32, (64, 32), 0)
    c = lax.broadcasted_iota(jnp.int32, (64, 32), 1)
    ecl = (r // 2 == c).astype(jnp.float32)
    r64 = lax.broadcasted_iota(jnp.int32, (64, 64), 0)
    c64 = lax.broadcasted_iota(jnp.int32, (64, 64), 1)
    sign = jnp.where(r64 % 2 == 0, -1.0, 1.0)
    pl_mat = jnp.where(c64 == (r64 ^ 1), sign, 0.0).astype(jnp.float32)
    return ecl, pl_mat


def _rot_body2(prev_ref, cs2_ref, x_ref, o_ref):
    del prev_ref
    _rot_core(cs2_ref, x_ref, o_ref)


def _rot_body(cs2_ref, x_ref, o_ref):
    _rot_core(cs2_ref, x_ref, o_ref)


def _rot_core(cs2_ref, x_ref, o_ref):
    cs2 = cs2_ref[...]
    cst = jnp.transpose(cs2)
    cs = cst[:64]
    ecl, pl_mat = _expand_mats()
    csx = jnp.dot(ecl, cs[:32],
                  preferred_element_type=jnp.float32,
                  precision=lax.Precision.HIGHEST)
    snx = jnp.dot(ecl, cs[32:],
                  preferred_element_type=jnp.float32,
                  precision=lax.Precision.HIGHEST)
    n = x_ref.shape[0]
    for i in range(n):
        xi = x_ref[i]
        rot = jnp.dot(pl_mat, xi,
                      preferred_element_type=jnp.float32,
                      precision=lax.Precision.DEFAULT)
        o_ref[i] = xi * csx + rot * snx


def kernel(positions, x_TNH, cache):
    T, N, H = x_TNH.shape
    V = cache.shape[0]
    NW = 32
    cache_t = jnp.transpose(cache)
    pos_idx = positions.reshape(NW, T // (NW * _CHUNK), _CHUNK)
    x_t = jnp.transpose(x_TNH, (1, 2, 0))

    TBC = 16384
    prep = pl.pallas_call(
        _prep_body,
        grid=(V // TBC,),
        in_specs=[pl.BlockSpec((H, TBC), lambda i: (0, i))],
        out_specs=pl.BlockSpec((TBC, 2 * H), lambda i: (i, 0)),
        out_shape=jax.ShapeDtypeStruct((V, 2 * H), jnp.float32),
    )
    src = prep(cache_t)

    half = T // 2
    gather = _make_gather(V, half)
    pos0 = positions[:half].reshape(NW, half // (NW * _CHUNK), _CHUNK)
    pos1 = positions[half:].reshape(NW, half // (NW * _CHUNK), _CHUNK)
    g0 = gather(pos0, src)
    g1 = gather(pos1, src)

    TB = 2048
    g2 = half // TB
    out_shape = jax.ShapeDtypeStruct((N, H, T), jnp.float32)
    rotate0 = pl.pallas_call(
        _rot_body,
        grid=(g2,),
        in_specs=[
            pl.BlockSpec((TB, 2 * H), lambda i: (i, 0)),
            pl.BlockSpec((N, H, TB), lambda i: (0, 0, i)),
        ],
        out_specs=pl.BlockSpec((N, H, TB), lambda i: (0, 0, i)),
        out_shape=out_shape,
    )
    out_a = rotate0(g0, x_t)
    rotate1 = pl.pallas_call(
        _rot_body2,
        grid=(g2,),
        in_specs=[
            pl.BlockSpec(memory_space=pl.ANY),
            pl.BlockSpec((TB, 2 * H), lambda i: (i, 0)),
            pl.BlockSpec((N, H, TB), lambda i: (0, 0, i + g2)),
        ],
        out_specs=pl.BlockSpec((N, H, TB), lambda i: (0, 0, i + g2)),
        out_shape=out_shape,
        input_output_aliases={0: 0},
    )
    out_t = rotate1(out_a, g1, x_t)
    return jnp.transpose(out_t, (2, 0, 1))


# speedup vs baseline: 1.0238x; 1.0238x over previous
import functools

import jax
import jax.numpy as jnp
from jax import lax
from jax.experimental import pallas as pl
from jax.experimental.pallas import tpu as pltpu
from jax.experimental.pallas import tpu_sc as plsc

_CHUNK = 128


def _prep_body(ct_ref, o_ref):
    ct = ct_ref[...]
    rows = jnp.transpose(ct)
    o_ref[...] = jnp.concatenate(
        [rows, jnp.zeros_like(rows)], axis=1)


def _make_gather(V, T):
    info = plsc.get_sparse_core_info()
    NC, NS = info.num_cores, info.num_subcores
    NW = NC * NS
    b_per_w = T // NW
    n_chunks = b_per_w // _CHUNK
    mesh = plsc.VectorSubcoreMesh(core_axis_name="c", subcore_axis_name="s")

    @functools.partial(
        pl.kernel,
        mesh=mesh,
        out_type=jax.ShapeDtypeStruct((T, 128), jnp.float32),
        scratch_types=[
            pltpu.VMEM((n_chunks, _CHUNK), jnp.int32),
            pltpu.VMEM((6, _CHUNK, 128), jnp.float32),
        ] + [pltpu.SemaphoreType.DMA] * 12,
    )
    def gather_k(pos_hbm, src_hbm, out_hbm, idx_v, rows_v, *sems):
        wid = lax.axis_index("s") * NC + lax.axis_index("c")
        base = wid * b_per_w
        pltpu.sync_copy(pos_hbm.at[wid], idx_v)
        rsems, wsems = sems[:6], sems[6:]
        nbuf, depth = 6, 4
        rh = [None] * nbuf
        wh = [None] * nbuf
        for j in range(n_chunks):
            b = j % nbuf
            if wh[b] is not None:
                wh[b].wait()
            if j >= depth:
                jj = j - depth
                bb = jj % nbuf
                rh[bb].wait()
                wh[bb] = pltpu.async_copy(
                    rows_v.at[bb],
                    out_hbm.at[pl.ds(base + jj * _CHUNK, _CHUNK)],
                    wsems[bb],
                )
            rh[b] = pltpu.async_copy(
                src_hbm.at[idx_v.at[j]], rows_v.at[b], rsems[b]
            )
        for jj in range(max(0, n_chunks - depth), n_chunks):
            bb = jj % nbuf
            rh[bb].wait()
            wh[bb] = pltpu.async_copy(
                rows_v.at[bb],
                out_hbm.at[pl.ds(base + jj * _CHUNK, _CHUNK)],
                wsems[bb],
            )
        for b in range(nbuf):
            if wh[b] is not None:
                wh[b].wait()

    return gather_k


def _expand_mats():
    r = lax.broadcasted_iota(jnp.int32, (64, 32), 0)
    c = lax.broadcasted_iota(jnp.int32, (64, 32), 1)
    ecl = (r // 2 == c).astype(jnp.float32)
    r64 = lax.broadcasted_iota(jnp.int32, (64, 64), 0)
    c64 = lax.broadcasted_iota(jnp.int32, (64, 64), 1)
    sign = jnp.where(r64 % 2 == 0, -1.0, 1.0)
    pl_mat = jnp.where(c64 == (r64 ^ 1), sign, 0.0).astype(jnp.float32)
    return ecl, pl_mat


def _rot_body(cs2_ref, x_ref, o_ref):
    cs2 = cs2_ref[...]
    cst = jnp.transpose(cs2)
    cs = cst[:64]
    ecl, pl_mat = _expand_mats()
    csx = jnp.dot(ecl, cs[:32],
                  preferred_element_type=jnp.float32,
                  precision=lax.Precision.HIGHEST)
    snx = jnp.dot(ecl, cs[32:],
                  preferred_element_type=jnp.float32,
                  precision=lax.Precision.HIGHEST)
    n = x_ref.shape[0]
    for i in range(n):
        xi = x_ref[i]
        rot = jnp.dot(pl_mat, xi,
                      preferred_element_type=jnp.float32,
                      precision=lax.Precision.DEFAULT)
        o_ref[i] = xi * csx + rot * snx


def kernel(positions, x_TNH, cache):
    T, N, H = x_TNH.shape
    V = cache.shape[0]
    NW = 32
    cache_t = jnp.transpose(cache)
    pos_idx = positions.reshape(NW, T // (NW * _CHUNK), _CHUNK)
    x_t = jnp.transpose(x_TNH, (1, 2, 0))

    TBC = 16384
    prep = pl.pallas_call(
        _prep_body,
        grid=(V // TBC,),
        in_specs=[pl.BlockSpec((H, TBC), lambda i: (0, i))],
        out_specs=pl.BlockSpec((TBC, 2 * H), lambda i: (i, 0)),
        out_shape=jax.ShapeDtypeStruct((V, 2 * H), jnp.float32),
    )
    src = prep(cache_t)

    cs2_TH = _make_gather(V, T)(pos_idx, src)

    TB = 2048
    rotate = pl.pallas_call(
        _rot_body,
        grid=(T // TB,),
        in_specs=[
            pl.BlockSpec((TB, 2 * H), lambda i: (i, 0)),
            pl.BlockSpec((N, H, TB), lambda i: (0, 0, i)),
        ],
        out_specs=pl.BlockSpec((N, H, TB), lambda i: (0, 0, i)),
        out_shape=jax.ShapeDtypeStruct((N, H, T), jnp.float32),
    )
    out_t = rotate(cs2_TH, x_t)
    return jnp.transpose(out_t, (2, 0, 1))


# speedup vs baseline: 1.0313x; 1.0073x over previous
import functools

import jax
import jax.numpy as jnp
from jax import lax
from jax.experimental import pallas as pl
from jax.experimental.pallas import tpu as pltpu
from jax.experimental.pallas import tpu_sc as plsc

_CHUNK = 128


def _prep_body(ct_ref, o_ref):
    ct = ct_ref[...]
    rows = jnp.transpose(ct)
    o_ref[...] = jnp.concatenate(
        [rows, jnp.zeros_like(rows)], axis=1)


def _make_gather(V, T):
    info = plsc.get_sparse_core_info()
    NC, NS = info.num_cores, info.num_subcores
    NW = NC * NS
    b_per_w = T // NW
    n_chunks = b_per_w // _CHUNK
    mesh = plsc.VectorSubcoreMesh(core_axis_name="c", subcore_axis_name="s")

    @functools.partial(
        pl.kernel,
        mesh=mesh,
        out_type=jax.ShapeDtypeStruct((T, 128), jnp.float32),
        scratch_types=[
            pltpu.VMEM((n_chunks, _CHUNK), jnp.int32),
            pltpu.VMEM((6, _CHUNK, 128), jnp.float32),
        ] + [pltpu.SemaphoreType.DMA] * 12,
    )
    def gather_k(pos_hbm, src_hbm, out_hbm, idx_v, rows_v, *sems):
        wid = lax.axis_index("s") * NC + lax.axis_index("c")
        base = wid * b_per_w
        pltpu.sync_copy(pos_hbm.at[wid], idx_v)
        rsems, wsems = sems[:6], sems[6:]
        nbuf, depth = 6, 4
        rh = [None] * nbuf
        wh = [None] * nbuf
        for j in range(n_chunks):
            b = j % nbuf
            if wh[b] is not None:
                wh[b].wait()
            if j >= depth:
                jj = j - depth
                bb = jj % nbuf
                rh[bb].wait()
                wh[bb] = pltpu.async_copy(
                    rows_v.at[bb],
                    out_hbm.at[pl.ds(base + jj * _CHUNK, _CHUNK)],
                    wsems[bb],
                )
            rh[b] = pltpu.async_copy(
                src_hbm.at[idx_v.at[j]], rows_v.at[b], rsems[b]
            )
        for jj in range(max(0, n_chunks - depth), n_chunks):
            bb = jj % nbuf
            rh[bb].wait()
            wh[bb] = pltpu.async_copy(
                rows_v.at[bb],
                out_hbm.at[pl.ds(base + jj * _CHUNK, _CHUNK)],
                wsems[bb],
            )
        for b in range(nbuf):
            if wh[b] is not None:
                wh[b].wait()

    return gather_k


def _expand_mats():
    r = lax.broadcasted_iota(jnp.int32, (64, 32), 0)
    c = lax.broadcasted_iota(jnp.int32, (64, 32), 1)
    ecl = (r // 2 == c).astype(jnp.float32)
    r64 = lax.broadcasted_iota(jnp.int32, (64, 64), 0)
    c64 = lax.broadcasted_iota(jnp.int32, (64, 64), 1)
    sign = jnp.where(r64 % 2 == 0, -1.0, 1.0)
    pl_mat = jnp.where(c64 == (r64 ^ 1), sign, 0.0).astype(jnp.float32)
    return ecl, pl_mat


def _rot_body(cs2_ref, x_ref, o_ref):
    cs2 = cs2_ref[...]
    cst = jnp.transpose(cs2)
    cs = cst[:64]
    ecl, pl_mat = _expand_mats()
    csx = jnp.dot(ecl, cs[:32],
                  preferred_element_type=jnp.float32,
                  precision=lax.Precision.HIGHEST)
    snx = jnp.dot(ecl, cs[32:],
                  preferred_element_type=jnp.float32,
                  precision=lax.Precision.HIGHEST)
    n = x_ref.shape[0]
    for i in range(n):
        xi = x_ref[i]
        rot = jnp.dot(pl_mat, xi,
                      preferred_element_type=jnp.float32,
                      precision=lax.Precision.DEFAULT)
        o_ref[i] = xi * csx + rot * snx


def kernel(positions, x_TNH, cache):
    T, N, H = x_TNH.shape
    V = cache.shape[0]
    NW = 32
    cache_t = jnp.transpose(cache)
    pos_idx = positions.reshape(NW, T // (NW * _CHUNK), _CHUNK)
    x_t = jnp.transpose(x_TNH, (1, 2, 0))

    TBC = 32768
    prep = pl.pallas_call(
        _prep_body,
        grid=(V // TBC,),
        in_specs=[pl.BlockSpec((H, TBC), lambda i: (0, i))],
        out_specs=pl.BlockSpec((TBC, 2 * H), lambda i: (i, 0)),
        out_shape=jax.ShapeDtypeStruct((V, 2 * H), jnp.float32),
    )
    src = prep(cache_t)

    cs2_TH = _make_gather(V, T)(pos_idx, src)

    TB = 2048
    rotate = pl.pallas_call(
        _rot_body,
        grid=(T // TB,),
        in_specs=[
            pl.BlockSpec((TB, 2 * H), lambda i: (i, 0)),
            pl.BlockSpec((N, H, TB), lambda i: (0, 0, i)),
        ],
        out_specs=pl.BlockSpec((N, H, TB), lambda i: (0, 0, i)),
        out_shape=jax.ShapeDtypeStruct((N, H, T), jnp.float32),
    )
    out_t = rotate(cs2_TH, x_t)
    return jnp.transpose(out_t, (2, 0, 1))
